# baseline (speedup 1.0000x reference)
import jax
import jax.numpy as jnp
from jax.experimental import pallas as pl
from jax.experimental.pallas import tpu as pltpu

_K = 9
_G = 6
_C = 256
_N = 256
_BB = 32
_CAPS = 8


def _conv_body(x_ee, x_eo, x_oe, x_oo, w_ref, b_ref, o_ref):
    phases = ((x_ee, x_eo), (x_oe, x_oo))
    bb = x_ee.shape[4]
    m = _G * _G * bb
    acc = jnp.zeros((m, _N), jnp.float32) + b_ref[...]
    for ky in range(_K):
        py, dy = ky % 2, ky // 2
        for kx in range(_K):
            px, dx = kx % 2, kx // 2
            a = phases[py][px][dy:dy + _G, 0, dx:dx + _G, 0, :, :]
            w_tap = w_ref[ky, kx].astype(jnp.bfloat16)
            acc += jnp.dot(a.reshape(m, _C), w_tap,
                           preferred_element_type=jnp.float32)
    o_ref[...] = acc.reshape(_G, _G, bb, _N).astype(jnp.bfloat16)


def _squash_body(y_ref, s_ref, st_ref, o_ref):
    bb = y_ref.shape[2]
    v = y_ref[...].reshape(_G * _G, bb, _N)
    t = jnp.transpose(v, (1, 2, 0))
    u = t.reshape(bb, _N * _G * _G).astype(jnp.float32)
    u2 = u * u
    s = s_ref[...]
    st = st_ref[...]
    parts = []
    for k in range(9):
        sq = jnp.dot(u2[:, 1024 * k:1024 * (k + 1)], s,
                     preferred_element_type=jnp.float32)
        scale = sq / ((1.0 + sq) * jnp.sqrt(sq))
        parts.append(jnp.dot(scale, st, preferred_element_type=jnp.float32))
    o_ref[...] = u * jnp.concatenate(parts, axis=1)


def kernel(w_mat, b_row, x):
    bsz = x.shape[0]
    bb = _BB if bsz % _BB == 0 else 8
    xt = jnp.transpose(x, (2, 3, 0, 1)).astype(jnp.bfloat16)
    xt = jax.lax.optimization_barrier(xt)
    xq = xt.reshape(10, 2, 10, 2, bsz, _C)
    wq = w_mat.reshape(_K, _K, _C, _N)

    def _phase_spec(py, px):
        return pl.BlockSpec((10, 1, 10, 1, bb, _C),
                            lambda j, py=py, px=px: (0, py, 0, px, j, 0))

    y = pl.pallas_call(
        _conv_body,
        out_shape=jax.ShapeDtypeStruct((_G, _G, bsz, _N), jnp.bfloat16),
        grid=(bsz // bb,),
        in_specs=[
            _phase_spec(0, 0),
            _phase_spec(0, 1),
            _phase_spec(1, 0),
            _phase_spec(1, 1),
            pl.BlockSpec((_K, _K, _C, _N), lambda j: (0, 0, 0, 0)),
            pl.BlockSpec((1, _N), lambda j: (0, 0)),
        ],
        out_specs=pl.BlockSpec((_G, _G, bb, _N), lambda j: (0, 0, j, 0)),
        compiler_params=pltpu.CompilerParams(
            dimension_semantics=("parallel",),
            vmem_limit_bytes=48 * 1024 * 1024,
        ),
    )(xq, xq, xq, xq, wq, b_row)

    grp = (jnp.arange(1024, dtype=jnp.int32)[:, None] // _CAPS ==
           jnp.arange(1024 // _CAPS, dtype=jnp.int32)[None, :])
    s_mat = grp.astype(jnp.float32)
    bb_sq = 32 if bsz % 32 == 0 else bb
    out_sq = pl.pallas_call(
        _squash_body,
        out_shape=jax.ShapeDtypeStruct((bsz, _N * _G * _G), jnp.float32),
        grid=(bsz // bb_sq,),
        in_specs=[
            pl.BlockSpec((_G, _G, bb_sq, _N), lambda j: (0, 0, j, 0)),
            pl.BlockSpec((1024, 1024 // _CAPS), lambda j: (0, 0)),
            pl.BlockSpec((1024 // _CAPS, 1024), lambda j: (0, 0)),
        ],
        out_specs=pl.BlockSpec((bb_sq, _N * _G * _G), lambda j: (j, 0)),
        compiler_params=pltpu.CompilerParams(
            dimension_semantics=("parallel",),
        ),
    )(y, s_mat, s_mat.T)
    return out_sq.reshape(bsz, 1152, _CAPS)


# speedup vs baseline: 1.0081x; 1.0081x over previous
import jax
import jax.numpy as jnp
from jax.experimental import pallas as pl
from jax.experimental.pallas import tpu as pltpu

_K = 9
_G = 6
_C = 256
_N = 256
_BB = 16
_CAPS = 8


def _conv_body(x_ee, x_eo, x_oe, x_oo, w_ref, b_ref, o_ref):
    phases = ((x_ee, x_eo), (x_oe, x_oo))
    bb = x_ee.shape[4]
    m = _G * _G * bb
    acc = jnp.zeros((m, _N), jnp.float32) + b_ref[...]
    for ky in range(_K):
        py, dy = ky % 2, ky // 2
        for kx in range(_K):
            px, dx = kx % 2, kx // 2
            a = phases[py][px][dy:dy + _G, 0, dx:dx + _G, 0, :, :]
            w_tap = w_ref[ky, kx].astype(jnp.bfloat16)
            acc += jnp.dot(a.reshape(m, _C), w_tap,
                           preferred_element_type=jnp.float32)
    o_ref[...] = acc.reshape(_G, _G, bb, _N).astype(jnp.bfloat16)


def _squash_body(y_ref, s_ref, st_ref, o_ref):
    bb = y_ref.shape[2]
    v = y_ref[...].reshape(_G * _G, bb, _N)
    t = jnp.transpose(v, (1, 2, 0))
    u = t.reshape(bb, _N * _G * _G).astype(jnp.float32)
    u2 = u * u
    s = s_ref[...]
    st = st_ref[...]
    parts = []
    for k in range(9):
        sq = jnp.dot(u2[:, 1024 * k:1024 * (k + 1)], s,
                     preferred_element_type=jnp.float32)
        scale = sq / ((1.0 + sq) * jnp.sqrt(sq))
        parts.append(jnp.dot(scale, st, preferred_element_type=jnp.float32))
    o_ref[...] = u * jnp.concatenate(parts, axis=1)


def kernel(w_mat, b_row, x):
    bsz = x.shape[0]
    bb = _BB if bsz % _BB == 0 else 8
    xt = jnp.transpose(x, (2, 3, 0, 1)).astype(jnp.bfloat16)
    xt = jax.lax.optimization_barrier(xt)
    xq = xt.reshape(10, 2, 10, 2, bsz, _C)
    wq = w_mat.reshape(_K, _K, _C, _N)

    def _phase_spec(py, px):
        return pl.BlockSpec((10, 1, 10, 1, bb, _C),
                            lambda j, py=py, px=px: (0, py, 0, px, j, 0))

    y = pl.pallas_call(
        _conv_body,
        out_shape=jax.ShapeDtypeStruct((_G, _G, bsz, _N), jnp.bfloat16),
        grid=(bsz // bb,),
        in_specs=[
            _phase_spec(0, 0),
            _phase_spec(0, 1),
            _phase_spec(1, 0),
            _phase_spec(1, 1),
            pl.BlockSpec((_K, _K, _C, _N), lambda j: (0, 0, 0, 0)),
            pl.BlockSpec((1, _N), lambda j: (0, 0)),
        ],
        out_specs=pl.BlockSpec((_G, _G, bb, _N), lambda j: (0, 0, j, 0)),
        compiler_params=pltpu.CompilerParams(
            dimension_semantics=("parallel",),
            vmem_limit_bytes=48 * 1024 * 1024,
        ),
    )(xq, xq, xq, xq, wq, b_row)

    grp = (jnp.arange(1024, dtype=jnp.int32)[:, None] // _CAPS ==
           jnp.arange(1024 // _CAPS, dtype=jnp.int32)[None, :])
    s_mat = grp.astype(jnp.float32)
    bb_sq = 32 if bsz % 32 == 0 else bb
    out_sq = pl.pallas_call(
        _squash_body,
        out_shape=jax.ShapeDtypeStruct((bsz, _N * _G * _G), jnp.float32),
        grid=(bsz // bb_sq,),
        in_specs=[
            pl.BlockSpec((_G, _G, bb_sq, _N), lambda j: (0, 0, j, 0)),
            pl.BlockSpec((1024, 1024 // _CAPS), lambda j: (0, 0)),
            pl.BlockSpec((1024 // _CAPS, 1024), lambda j: (0, 0)),
        ],
        out_specs=pl.BlockSpec((bb_sq, _N * _G * _G), lambda j: (j, 0)),
        compiler_params=pltpu.CompilerParams(
            dimension_semantics=("parallel",),
        ),
    )(y, s_mat, s_mat.T)
    return out_sq.reshape(bsz, 1152, _CAPS)


# speedup vs baseline: 1.0267x; 1.0185x over previous
import jax
import jax.numpy as jnp
from jax.experimental import pallas as pl
from jax.experimental.pallas import tpu as pltpu

_K = 9
_G = 6
_C = 256
_N = 256
_BB = 16
_CAPS = 8


def _conv_body(x_ee, x_eo, x_oe, x_oo, w_ref, b_ref, o_ref):
    phases = ((x_ee, x_eo), (x_oe, x_oo))
    bb = x_ee.shape[4]
    m = _G * _G * bb
    acc = jnp.zeros((m, _N), jnp.float32) + b_ref[...]
    for ky in range(_K):
        py, dy = ky % 2, ky // 2
        for kx in range(_K):
            px, dx = kx % 2, kx // 2
            a = phases[py][px][dy:dy + _G, 0, dx:dx + _G, 0, :, :]
            w_tap = w_ref[ky, kx].astype(jnp.bfloat16)
            acc += jnp.dot(a.reshape(m, _C), w_tap,
                           preferred_element_type=jnp.float32)
    o_ref[...] = acc.reshape(_G, _G, bb, _N).astype(jnp.bfloat16)


def _squash_body(y_ref, s_ref, st_ref, o_ref):
    bb = y_ref.shape[2]
    v = y_ref[...].reshape(_G * _G, bb, _N)
    t = jnp.transpose(v, (1, 2, 0))
    u = t.reshape(bb, _N * _G * _G).astype(jnp.float32)
    u2 = u * u
    s = s_ref[...]
    st = st_ref[...]
    parts = []
    for k in range(9):
        sq = jnp.dot(u2[:, 1024 * k:1024 * (k + 1)], s,
                     preferred_element_type=jnp.float32)
        scale = sq / ((1.0 + sq) * jnp.sqrt(sq))
        parts.append(jnp.dot(scale, st, preferred_element_type=jnp.float32))
    o_ref[...] = u * jnp.concatenate(parts, axis=1)


def kernel(w_mat, b_row, x):
    bsz = x.shape[0]
    bb = _BB if bsz % _BB == 0 else 8
    xt = jnp.transpose(x, (2, 3, 0, 1)).astype(jnp.bfloat16)
    xt = jax.lax.optimization_barrier(xt)
    xq = xt.reshape(10, 2, 10, 2, bsz, _C)
    wq = w_mat.reshape(_K, _K, _C, _N)

    def _phase_spec(py, px):
        return pl.BlockSpec((10, 1, 10, 1, bb, _C),
                            lambda j, py=py, px=px: (0, py, 0, px, j, 0))

    y = pl.pallas_call(
        _conv_body,
        out_shape=jax.ShapeDtypeStruct((_G, _G, bsz, _N), jnp.bfloat16),
        grid=(bsz // bb,),
        in_specs=[
            _phase_spec(0, 0),
            _phase_spec(0, 1),
            _phase_spec(1, 0),
            _phase_spec(1, 1),
            pl.BlockSpec((_K, _K, _C, _N), lambda j: (0, 0, 0, 0)),
            pl.BlockSpec((1, _N), lambda j: (0, 0)),
        ],
        out_specs=pl.BlockSpec((_G, _G, bb, _N), lambda j: (0, 0, j, 0)),
        compiler_params=pltpu.CompilerParams(
            dimension_semantics=("parallel",),
            vmem_limit_bytes=48 * 1024 * 1024,
        ),
    )(xq, xq, xq, xq, wq, b_row)

    grp = (jnp.arange(1024, dtype=jnp.int32)[:, None] // _CAPS ==
           jnp.arange(1024 // _CAPS, dtype=jnp.int32)[None, :])
    s_mat = grp.astype(jnp.float32)
    bb_sq = 64 if bsz % 64 == 0 else (32 if bsz % 32 == 0 else bb)
    out_sq = pl.pallas_call(
        _squash_body,
        out_shape=jax.ShapeDtypeStruct((bsz, _N * _G * _G), jnp.float32),
        grid=(bsz // bb_sq,),
        in_specs=[
            pl.BlockSpec((_G, _G, bb_sq, _N), lambda j: (0, 0, j, 0)),
            pl.BlockSpec((1024, 1024 // _CAPS), lambda j: (0, 0)),
            pl.BlockSpec((1024 // _CAPS, 1024), lambda j: (0, 0)),
        ],
        out_specs=pl.BlockSpec((bb_sq, _N * _G * _G), lambda j: (j, 0)),
        compiler_params=pltpu.CompilerParams(
            dimension_semantics=("parallel",),
        ),
    )(y, s_mat, s_mat.T)
    return out_sq.reshape(bsz, 1152, _CAPS)
